# Initial kernel scaffold; baseline (speedup 1.0000x reference)
import jax
import jax.numpy as jnp
from jax.experimental import pallas as pl


def kernel(emb_table, W1, b1, W2, b2, laplacian_values, user, pos, neg, laplacian_indices):
    raise NotImplementedError("write your pallas kernel here")


# speedup vs baseline: 4.7570x; 4.7570x over previous
import functools

import jax
import jax.numpy as jnp
from jax import lax
from jax.experimental import pallas as pl
from jax.experimental.pallas import tpu as pltpu
from jax.experimental.pallas import tpu_sc as plsc

N = 50000
D = 64
L = 3
B = 4096
E = 800000
G = 3 * B

NC = 2
NS = 16
HALF = N // NC
STR = 1568
STR_LAST = HALF - 15 * STR

EPT = E // NS
SUB = 80
NBUF = 4
ITER_E = SUB * NBUF
BLK = 4 * ITER_E
OUTER = EPT // ITER_E
TAIL_OFF = OUTER * ITER_E

GW = G // (NC * NS)

_mesh = plsc.VectorSubcoreMesh(core_axis_name="c", subcore_axis_name="s")
_sc_params = pltpu.CompilerParams(use_tc_tiling_on_sc=False)


def _lane_bcast(v, i):
    idx = jnp.full((16, 1), i, jnp.int32)
    dn = lax.GatherDimensionNumbers(
        offset_dims=(), collapsed_slice_dims=(0,), start_index_map=(0,))
    return lax.gather(v, idx, dn, (1,),
                      mode=lax.GatherScatterMode.PROMISE_IN_BOUNDS)


def _mask_subchunk(c, rowb, valb, idxsb, b, off):
    for g in range(SUB // 16):
        sl = pl.ds(off + g * 16, 16)
        r = rowb[sl]
        loc = r - c * HALF
        inb = (loc >= 0) & (loc < HALF)
        idxsb[b, pl.ds(g * 16, 16)] = jnp.where(inb, loc, 0)
        valb[sl] = jnp.where(inb, valb[sl], 0.0)


def _scale_subchunk(valb, rowsb, b, off):
    for g in range(SUB // 16):
        vv = valb[pl.ds(off + g * 16, 16)]
        for j in range(16):
            bc = _lane_bcast(vv, j)
            i = g * 16 + j
            for q in range(4):
                sl = pl.ds(q * 16, 16)
                rowsb[b, i, sl] = rowsb[b, i, sl] * bc


def _gather_batch(emb_hbm, uidx_hbm, gath_hbm, gidx, rowsb, sem, wid):
    pltpu.sync_copy(uidx_hbm.at[pl.ds(wid * GW, GW)], gidx)
    offs_sizes = [(0, 80), (80, 80), (160, 80), (240, 80), (320, 64)]
    for k, (o, sz) in enumerate(offs_sizes):
        bsel = k % NBUF
        dstbuf = rowsb.at[bsel] if sz == SUB else rowsb.at[bsel].at[pl.ds(0, sz)]
        pltpu.async_copy(emb_hbm.at[gidx.at[pl.ds(o, sz)]], dstbuf, sem).wait()
        pltpu.sync_copy(dstbuf, gath_hbm.at[pl.ds(wid * GW + o, sz)])


def _spmm_body(emb_hbm, erow_hbm, ecol_hbm, eval_hbm, uidx_hbm,
               lap_hbm, gath_hbm,
               colb, rowb, valb, rowsb, idxsb, gidx,
               acc, gsem, ssem):
    c = lax.axis_index("c")
    s = lax.axis_index("s")

    @pl.loop(0, SUB)
    def _(i):
        z = jnp.zeros((16,), jnp.float32)
        for b in range(NBUF):
            for q in range(4):
                rowsb[b, i, pl.ds(q * 16, 16)] = z

    @pl.when(s < NS - 1)
    def _():
        zhs = [pltpu.async_copy(
                   rowsb.at[0], acc.at[pl.ds(s * STR + k * SUB, SUB)],
                   gsem.at[k % NBUF]) for k in range(19)]
        zhs.append(pltpu.async_copy(
            rowsb.at[1].at[pl.ds(0, 48)],
            acc.at[pl.ds(s * STR + 19 * SUB, 48)], gsem.at[3]))
        for h in zhs:
            h.wait()

    @pl.when(s == NS - 1)
    def _():
        zhs = [pltpu.async_copy(
                   rowsb.at[0], acc.at[pl.ds(15 * STR + k * SUB, SUB)],
                   gsem.at[k % NBUF]) for k in range(18)]
        zhs.append(pltpu.async_copy(
            rowsb.at[1].at[pl.ds(0, 40)],
            acc.at[pl.ds(15 * STR + 18 * SUB, 40)], gsem.at[3]))
        for h in zhs:
            h.wait()

    plsc.subcore_barrier()

    wid = s * NC + c
    _gather_batch(emb_hbm, uidx_hbm, gath_hbm, gidx, rowsb, gsem.at[0], wid)

    @pl.loop(0, OUTER)
    def _(t):
        @pl.when(t % 4 == 0)
        def _():
            eb = s * EPT + (t // 4) * BLK
            pltpu.sync_copy(ecol_hbm.at[pl.ds(eb, BLK)], colb)
            pltpu.sync_copy(erow_hbm.at[pl.ds(eb, BLK)], rowb)
            pltpu.sync_copy(eval_hbm.at[pl.ds(eb, BLK)], valb)

        boff = (t % 4) * ITER_E
        ghs = [pltpu.async_copy(
                   emb_hbm.at[colb.at[pl.ds(boff + b * SUB, SUB)]],
                   rowsb.at[b], gsem.at[b]) for b in range(NBUF)]
        shs = []
        for b in range(NBUF):
            off = boff + b * SUB
            _mask_subchunk(c, rowb, valb, idxsb, b, off)
            ghs[b].wait()
            _scale_subchunk(valb, rowsb, b, off)
            shs.append(pltpu.async_copy(rowsb.at[b], acc.at[idxsb.at[b]],
                                        ssem.at[b], add=True))
        for h in shs:
            h.wait()

    tb = s * EPT + TAIL_OFF
    pltpu.sync_copy(ecol_hbm.at[pl.ds(tb, SUB)], colb.at[pl.ds(0, SUB)])
    pltpu.sync_copy(erow_hbm.at[pl.ds(tb, SUB)], rowb.at[pl.ds(0, SUB)])
    pltpu.sync_copy(eval_hbm.at[pl.ds(tb, SUB)], valb.at[pl.ds(0, SUB)])
    gh = pltpu.async_copy(emb_hbm.at[colb.at[pl.ds(0, SUB)]],
                          rowsb.at[0], gsem.at[0])
    _mask_subchunk(c, rowb, valb, idxsb, 0, 0)
    gh.wait()
    _scale_subchunk(valb, rowsb, 0, 0)
    pltpu.sync_copy(rowsb.at[0], acc.at[idxsb.at[0]], add=True)

    plsc.subcore_barrier()

    @pl.when(s < NS - 1)
    def _():
        pltpu.async_copy(acc.at[pl.ds(s * STR, STR)],
                         lap_hbm.at[pl.ds(c * HALF + s * STR, STR)],
                         gsem.at[0]).wait()

    @pl.when(s == NS - 1)
    def _():
        pltpu.async_copy(acc.at[pl.ds(15 * STR, STR_LAST)],
                         lap_hbm.at[pl.ds(c * HALF + 15 * STR, STR_LAST)],
                         gsem.at[0]).wait()


_spmm_call = functools.partial(
    pl.kernel,
    out_type=[jax.ShapeDtypeStruct((N, D), jnp.float32),
              jax.ShapeDtypeStruct((G, D), jnp.float32)],
    mesh=_mesh,
    compiler_params=_sc_params,
    scratch_types=[
        pltpu.VMEM((BLK,), jnp.int32),
        pltpu.VMEM((BLK,), jnp.int32),
        pltpu.VMEM((BLK,), jnp.float32),
        pltpu.VMEM((NBUF, SUB, D), jnp.float32),
        pltpu.VMEM((NBUF, SUB), jnp.int32),
        pltpu.VMEM((GW,), jnp.int32),
        pltpu.VMEM_SHARED((HALF, D), jnp.float32),
        pltpu.SemaphoreType.DMA((NBUF,)),
        pltpu.SemaphoreType.DMA((NBUF,)),
    ],
)(_spmm_body)


def _gather_body(emb_hbm, uidx_hbm, out_hbm, gidx, rowsb, sem):
    c = lax.axis_index("c")
    s = lax.axis_index("s")
    wid = s * NC + c
    _gather_batch(emb_hbm, uidx_hbm, out_hbm, gidx, rowsb, sem, wid)


_gather_call = functools.partial(
    pl.kernel,
    out_type=jax.ShapeDtypeStruct((G, D), jnp.float32),
    mesh=_mesh,
    compiler_params=_sc_params,
    scratch_types=[
        pltpu.VMEM((GW,), jnp.int32),
        pltpu.VMEM((NBUF, SUB, D), jnp.float32),
        pltpu.SemaphoreType.DMA,
    ],
)(_gather_body)


TB = 2000


def _transform_body(emb_ref, lap_ref, w1_ref, b1_ref, w2_ref, b2_ref, out_ref):
    e = emb_ref[...]
    la = lap_ref[...]
    sx = jnp.dot(la + e, w1_ref[...], preferred_element_type=jnp.float32) \
        + b1_ref[...]
    ox = la * (jnp.dot(e, w2_ref[...], preferred_element_type=jnp.float32)
               + b2_ref[...])
    x = sx + ox
    out_ref[...] = jnp.where(x >= 0, x, 0.01 * x)


def _transform(emb, lap, w1t, b1l, w2t, b2l):
    return pl.pallas_call(
        _transform_body,
        grid=(N // TB,),
        in_specs=[
            pl.BlockSpec((TB, D), lambda i: (i, 0)),
            pl.BlockSpec((TB, D), lambda i: (i, 0)),
            pl.BlockSpec((D, D), lambda i: (0, 0)),
            pl.BlockSpec((1, D), lambda i: (0, 0)),
            pl.BlockSpec((D, D), lambda i: (0, 0)),
            pl.BlockSpec((1, D), lambda i: (0, 0)),
        ],
        out_specs=pl.BlockSpec((TB, D), lambda i: (i, 0)),
        out_shape=jax.ShapeDtypeStruct((N, D), jnp.float32),
    )(emb, lap, w1t, b1l.reshape(1, D), w2t, b2l.reshape(1, D))


def _loss_body(r0, r1, r2, r3, out_ref):
    pos = jnp.zeros((B, 1), jnp.float32)
    neg = jnp.zeros((B, 1), jnp.float32)
    for r in (r0, r1, r2, r3):
        u = r[0:B, :]
        p = r[B:2 * B, :]
        n = r[2 * B:3 * B, :]
        pos = pos + jnp.sum(u * p, axis=1, keepdims=True)
        neg = neg + jnp.sum(u * n, axis=1, keepdims=True)
    x = pos - neg
    loss = jnp.maximum(-x, 0.0) + jnp.log1p(jnp.exp(-jnp.abs(x)))
    out_ref[...] = jnp.sum(loss).reshape(1, 1)


def _loss(g0, g1, g2, g3):
    out = pl.pallas_call(
        _loss_body,
        out_shape=jax.ShapeDtypeStruct((1, 1), jnp.float32),
    )(g0, g1, g2, g3)
    return out[0, 0]


def kernel(emb_table, W1, b1, W2, b2, laplacian_values, user, pos, neg,
           laplacian_indices):
    erow = laplacian_indices[0]
    ecol = laplacian_indices[1]
    uidx = jnp.concatenate([user, pos, neg]).astype(jnp.int32)
    emb = emb_table
    gs = []
    for l in range(L):
        lap, gath = _spmm_call(emb, erow, ecol, laplacian_values, uidx)
        gs.append(gath)
        emb = _transform(emb, lap, W1[l].T, b1[l], W2[l].T, b2[l])
    gs.append(_gather_call(emb, uidx))
    return _loss(*gs)


# speedup vs baseline: 7.2905x; 1.5326x over previous
import functools

import jax
import jax.numpy as jnp
from jax import lax
from jax.experimental import pallas as pl
from jax.experimental.pallas import tpu as pltpu
from jax.experimental.pallas import tpu_sc as plsc

N = 50000
D = 64
L = 3
B = 4096
E = 800000
G = 3 * B

NC = 2
NS = 16
NW = NC * NS
HALF = N // NC
STR = 1568
STR_LAST = HALF - 15 * STR

EPT = E // NS
SUB = 80
NBUF = 4
ITER_E = SUB * NBUF
BLK = 4 * ITER_E
CAP = 51200

PBLK = 2000
PCH = EPT // 16

GW = G // NW

_mesh = plsc.VectorSubcoreMesh(core_axis_name="c", subcore_axis_name="s")
_sc_params = pltpu.CompilerParams(use_tc_tiling_on_sc=False,
                                  needs_layout_passes=False)


def _lane_bcast(v, i):
    idx = jnp.full((16, 1), i, jnp.int32)
    dn = lax.GatherDimensionNumbers(
        offset_dims=(), collapsed_slice_dims=(0,), start_index_map=(0,))
    return lax.gather(v, idx, dn, (1,),
                      mode=lax.GatherScatterMode.PROMISE_IN_BOUNDS)


def _pre_body(erow_hbm, ecol_hbm, eval_hbm,
              cpk_hbm, cval_hbm, cnt_hbm,
              rb, cb, vb, pkst, vlst, cntb):
    c = lax.axis_index("c")
    s = lax.axis_index("s")
    w = s * NC + c

    def chunk(i, cnt):
        @pl.when(i % (PBLK // 16) == 0)
        def _():
            eb = s * EPT + (i // (PBLK // 16)) * PBLK
            pltpu.sync_copy(erow_hbm.at[pl.ds(eb, PBLK)], rb)
            pltpu.sync_copy(ecol_hbm.at[pl.ds(eb, PBLK)], cb)
            pltpu.sync_copy(eval_hbm.at[pl.ds(eb, PBLK)], vb)

        off = (i % (PBLK // 16)) * 16
        sl = pl.ds(off, 16)
        r = rb[sl]
        col = cb[sl]
        v = vb[sl]
        loc = r - c * HALF
        inb = (loc >= 0) & (loc < HALF)
        pk = jnp.bitwise_or(lax.shift_left(loc, 16), col)
        plsc.store_compressed(pkst.at[pl.ds(cnt, 16)], pk, mask=inb)
        plsc.store_compressed(vlst.at[pl.ds(cnt, 16)], v, mask=inb)
        pc = jnp.max(plsc.all_reduce_population_count(inb))
        return cnt + pc

    cnt = lax.fori_loop(0, PCH, chunk, jnp.int32(0))

    z32 = jnp.zeros((16,), jnp.int32)
    zf = jnp.zeros((16,), jnp.float32)
    rem = lax.rem(cnt, 16)
    fill = jnp.arange(16, dtype=jnp.int32) < (16 - rem)
    plsc.store_compressed(pkst.at[pl.ds(cnt, 16)], z32, mask=fill)
    plsc.store_compressed(vlst.at[pl.ds(cnt, 16)], zf, mask=fill)
    cnt16 = cnt + lax.rem(16 - rem, 16)
    for k in range(ITER_E // 16):
        pkst[pl.ds(cnt16 + k * 16, 16)] = z32
        vlst[pl.ds(cnt16 + k * 16, 16)] = zf
    n_iter = lax.div(cnt + ITER_E - 1, ITER_E)
    cntb[pl.ds(0, 16)] = jnp.full((16,), n_iter, jnp.int32)
    pltpu.sync_copy(cntb, cnt_hbm.at[w])
    pltpu.sync_copy(pkst, cpk_hbm.at[w])
    pltpu.sync_copy(vlst, cval_hbm.at[w])


_pre_call = functools.partial(
    pl.kernel,
    out_type=[jax.ShapeDtypeStruct((NW, CAP), jnp.int32),
              jax.ShapeDtypeStruct((NW, CAP), jnp.float32),
              jax.ShapeDtypeStruct((NW, 16), jnp.int32)],
    mesh=_mesh,
    compiler_params=_sc_params,
    scratch_types=[
        pltpu.VMEM((PBLK,), jnp.int32),
        pltpu.VMEM((PBLK,), jnp.int32),
        pltpu.VMEM((PBLK,), jnp.float32),
        pltpu.VMEM((CAP,), jnp.int32),
        pltpu.VMEM((CAP,), jnp.float32),
        pltpu.VMEM((16,), jnp.int32),
    ],
)(_pre_body)


def _scale_subchunk(vlb, rowsb, b, off):
    for g in range(SUB // 16):
        vv = vlb[pl.ds(off + g * 16, 16)]
        for j in range(16):
            bc = _lane_bcast(vv, j)
            i = g * 16 + j
            for q in range(4):
                sl = pl.ds(q * 16, 16)
                rowsb[b, i, sl] = rowsb[b, i, sl] * bc


def _gather_batch(emb_hbm, uidx_hbm, gath_hbm, gidx, rowsb, sem, wid):
    pltpu.sync_copy(uidx_hbm.at[pl.ds(wid * GW, GW)], gidx)
    offs_sizes = [(0, 80), (80, 80), (160, 80), (240, 80), (320, 64)]
    for k, (o, sz) in enumerate(offs_sizes):
        bsel = k % NBUF
        dstbuf = rowsb.at[bsel] if sz == SUB else rowsb.at[bsel].at[pl.ds(0, sz)]
        pltpu.async_copy(emb_hbm.at[gidx.at[pl.ds(o, sz)]], dstbuf, sem).wait()
        pltpu.sync_copy(dstbuf, gath_hbm.at[pl.ds(wid * GW + o, sz)])


def _spmm_body(emb_hbm, cpk_hbm, cval_hbm, cnt_hbm, uidx_hbm,
               lap_hbm, gath_hbm,
               pkb, vlb, rowsb, idxgb, idxsb, gidx, cntb,
               acc, gsem, ssem):
    c = lax.axis_index("c")
    s = lax.axis_index("s")
    w = s * NC + c

    @pl.loop(0, SUB)
    def _(i):
        z = jnp.zeros((16,), jnp.float32)
        for b in range(NBUF):
            for q in range(4):
                rowsb[b, i, pl.ds(q * 16, 16)] = z

    @pl.when(s < NS - 1)
    def _():
        zhs = [pltpu.async_copy(
                   rowsb.at[0], acc.at[pl.ds(s * STR + k * SUB, SUB)],
                   gsem.at[k % NBUF]) for k in range(19)]
        zhs.append(pltpu.async_copy(
            rowsb.at[1].at[pl.ds(0, 48)],
            acc.at[pl.ds(s * STR + 19 * SUB, 48)], gsem.at[3]))
        for h in zhs:
            h.wait()

    @pl.when(s == NS - 1)
    def _():
        zhs = [pltpu.async_copy(
                   rowsb.at[0], acc.at[pl.ds(15 * STR + k * SUB, SUB)],
                   gsem.at[k % NBUF]) for k in range(18)]
        zhs.append(pltpu.async_copy(
            rowsb.at[1].at[pl.ds(0, 40)],
            acc.at[pl.ds(15 * STR + 18 * SUB, 40)], gsem.at[3]))
        for h in zhs:
            h.wait()

    plsc.subcore_barrier()

    _gather_batch(emb_hbm, uidx_hbm, gath_hbm, gidx, rowsb, gsem.at[0], w)

    pltpu.sync_copy(cnt_hbm.at[w], cntb)
    n_iter = jnp.max(cntb[pl.ds(0, 16)])

    def iter_body(t, carry):
        @pl.when(t % 4 == 0)
        def _():
            eb = (t // 4) * BLK
            pltpu.sync_copy(cpk_hbm.at[w].at[pl.ds(eb, BLK)], pkb)
            pltpu.sync_copy(cval_hbm.at[w].at[pl.ds(eb, BLK)], vlb)

        @pl.when(t > 0)
        def _():
            for b in range(NBUF):
                pltpu.make_async_copy(rowsb.at[b], acc.at[idxsb.at[b]],
                                      ssem.at[b]).wait()

        boff = (t % 4) * ITER_E
        ghs = []
        for b in range(NBUF):
            off = boff + b * SUB
            for g in range(SUB // 16):
                pk = pkb[pl.ds(off + g * 16, 16)]
                gsl = pl.ds(g * 16, 16)
                idxgb[b, gsl] = jnp.bitwise_and(pk, 0xFFFF)
                idxsb[b, gsl] = lax.shift_right_logical(pk, 16)
            ghs.append(pltpu.async_copy(emb_hbm.at[idxgb.at[b]],
                                        rowsb.at[b], gsem.at[b]))
        for b in range(NBUF):
            ghs[b].wait()
            _scale_subchunk(vlb, rowsb, b, boff + b * SUB)
            pltpu.async_copy(rowsb.at[b], acc.at[idxsb.at[b]],
                             ssem.at[b], add=True)
        return carry

    lax.fori_loop(0, n_iter, iter_body, jnp.int32(0))

    @pl.when(n_iter > 0)
    def _():
        for b in range(NBUF):
            pltpu.make_async_copy(rowsb.at[b], acc.at[idxsb.at[b]],
                                  ssem.at[b]).wait()

    plsc.subcore_barrier()

    @pl.when(s < NS - 1)
    def _():
        pltpu.async_copy(acc.at[pl.ds(s * STR, STR)],
                         lap_hbm.at[pl.ds(c * HALF + s * STR, STR)],
                         gsem.at[0]).wait()

    @pl.when(s == NS - 1)
    def _():
        pltpu.async_copy(acc.at[pl.ds(15 * STR, STR_LAST)],
                         lap_hbm.at[pl.ds(c * HALF + 15 * STR, STR_LAST)],
                         gsem.at[0]).wait()


_spmm_call = functools.partial(
    pl.kernel,
    out_type=[jax.ShapeDtypeStruct((N, D), jnp.float32),
              jax.ShapeDtypeStruct((G, D), jnp.float32)],
    mesh=_mesh,
    compiler_params=_sc_params,
    scratch_types=[
        pltpu.VMEM((BLK,), jnp.int32),
        pltpu.VMEM((BLK,), jnp.float32),
        pltpu.VMEM((NBUF, SUB, D), jnp.float32),
        pltpu.VMEM((NBUF, SUB), jnp.int32),
        pltpu.VMEM((NBUF, SUB), jnp.int32),
        pltpu.VMEM((GW,), jnp.int32),
        pltpu.VMEM((16,), jnp.int32),
        pltpu.VMEM_SHARED((HALF, D), jnp.float32),
        pltpu.SemaphoreType.DMA((NBUF,)),
        pltpu.SemaphoreType.DMA((NBUF,)),
    ],
)(_spmm_body)


def _gather_body(emb_hbm, uidx_hbm, out_hbm, gidx, rowsb, sem):
    c = lax.axis_index("c")
    s = lax.axis_index("s")
    wid = s * NC + c
    _gather_batch(emb_hbm, uidx_hbm, out_hbm, gidx, rowsb, sem, wid)


_gather_call = functools.partial(
    pl.kernel,
    out_type=jax.ShapeDtypeStruct((G, D), jnp.float32),
    mesh=_mesh,
    compiler_params=_sc_params,
    scratch_types=[
        pltpu.VMEM((GW,), jnp.int32),
        pltpu.VMEM((NBUF, SUB, D), jnp.float32),
        pltpu.SemaphoreType.DMA,
    ],
)(_gather_body)


TB = 2000


def _transform_body(emb_ref, lap_ref, w1_ref, b1_ref, w2_ref, b2_ref, out_ref):
    e = emb_ref[...]
    la = lap_ref[...]
    sx = jnp.dot(la + e, w1_ref[...], preferred_element_type=jnp.float32) \
        + b1_ref[...]
    ox = la * (jnp.dot(e, w2_ref[...], preferred_element_type=jnp.float32)
               + b2_ref[...])
    x = sx + ox
    out_ref[...] = jnp.where(x >= 0, x, 0.01 * x)


def _transform(emb, lap, w1t, b1l, w2t, b2l):
    return pl.pallas_call(
        _transform_body,
        grid=(N // TB,),
        in_specs=[
            pl.BlockSpec((TB, D), lambda i: (i, 0)),
            pl.BlockSpec((TB, D), lambda i: (i, 0)),
            pl.BlockSpec((D, D), lambda i: (0, 0)),
            pl.BlockSpec((1, D), lambda i: (0, 0)),
            pl.BlockSpec((D, D), lambda i: (0, 0)),
            pl.BlockSpec((1, D), lambda i: (0, 0)),
        ],
        out_specs=pl.BlockSpec((TB, D), lambda i: (i, 0)),
        out_shape=jax.ShapeDtypeStruct((N, D), jnp.float32),
    )(emb, lap, w1t, b1l.reshape(1, D), w2t, b2l.reshape(1, D))


def _loss_body(r0, r1, r2, r3, out_ref):
    pos = jnp.zeros((B, 1), jnp.float32)
    neg = jnp.zeros((B, 1), jnp.float32)
    for r in (r0, r1, r2, r3):
        u = r[0:B, :]
        p = r[B:2 * B, :]
        n = r[2 * B:3 * B, :]
        pos = pos + jnp.sum(u * p, axis=1, keepdims=True)
        neg = neg + jnp.sum(u * n, axis=1, keepdims=True)
    x = pos - neg
    loss = jnp.maximum(-x, 0.0) + jnp.log1p(jnp.exp(-jnp.abs(x)))
    out_ref[...] = jnp.sum(loss).reshape(1, 1)


def _loss(g0, g1, g2, g3):
    out = pl.pallas_call(
        _loss_body,
        out_shape=jax.ShapeDtypeStruct((1, 1), jnp.float32),
    )(g0, g1, g2, g3)
    return out[0, 0]


def kernel(emb_table, W1, b1, W2, b2, laplacian_values, user, pos, neg,
           laplacian_indices):
    erow = laplacian_indices[0]
    ecol = laplacian_indices[1]
    uidx = jnp.concatenate([user, pos, neg]).astype(jnp.int32)
    cpk, cval, ccnt = _pre_call(erow, ecol, laplacian_values)
    emb = emb_table
    gs = []
    for l in range(L):
        lap, gath = _spmm_call(emb, cpk, cval, ccnt, uidx)
        gs.append(gath)
        emb = _transform(emb, lap, W1[l].T, b1[l], W2[l].T, b2[l])
    gs.append(_gather_call(emb, uidx))
    return _loss(*gs)
